# Initial kernel scaffold; baseline (speedup 1.0000x reference)
import jax
import jax.numpy as jnp
from jax.experimental import pallas as pl


def kernel(nodes, nodes_l1paths, nodes_l1n_attrs, u2e, v2e, r2e, ua2e, va2e, w1_w, w1_b, w2_w, w2_b, a1_w, a1_b, a2_w, a2_b, a3_w, a3_b):
    raise NotImplementedError("write your pallas kernel here")


# speedup vs baseline: 3.0742x; 3.0742x over previous
import functools

import jax
import jax.numpy as jnp
from jax import lax
from jax.experimental import pallas as pl
from jax.experimental.pallas import tpu as pltpu
from jax.experimental.pallas import tpu_sc as plsc

B, K, A, D = 1024, 32, 8, 128
BK = B * K

NC, NS = 2, 16
NW = NC * NS
SLOTS_W = BK // NW
CH = 256
N_CH = SLOTS_W // CH
ACH_S = 16
ACH_R = ACH_S * A
N_ACH = SLOTS_W // ACH_S
SELF_W = B // NW


def _sc_gather_body(neigh_hbm, rel_hbm, attr_hbm, node_hbm,
                    v2e_hbm, r2e_hbm, va2e_hbm, u2e_hbm,
                    n_out, r_out, a_out, s_out,
                    idx_n, idx_r, idx_a, idx_s,
                    bufn, bufr, abuf, sbuf, sebuf, sem):
    wid = lax.axis_index("s") * NC + lax.axis_index("c")
    base = wid * SLOTS_W

    sbase = wid * SELF_W
    pltpu.sync_copy(node_hbm.at[pl.ds(sbase, SELF_W)], idx_s)
    pltpu.async_copy(u2e_hbm.at[idx_s], sebuf, sem).wait()
    pltpu.sync_copy(sebuf, s_out.at[pl.ds(sbase, SELF_W)])

    def nr_body(c, carry):
        rb = base + c * CH
        pltpu.sync_copy(neigh_hbm.at[pl.ds(rb, CH)], idx_n)
        pltpu.async_copy(v2e_hbm.at[idx_n], bufn, sem).wait()
        pltpu.sync_copy(bufn, n_out.at[pl.ds(rb, CH)])
        pltpu.sync_copy(rel_hbm.at[pl.ds(rb, CH)], idx_r)
        pltpu.async_copy(r2e_hbm.at[idx_r], bufr, sem).wait()
        pltpu.sync_copy(bufr, r_out.at[pl.ds(rb, CH)])
        return carry

    lax.fori_loop(0, N_CH, nr_body, 0)

    def a_body(c, carry):
        sb = base + c * ACH_S
        pltpu.sync_copy(attr_hbm.at[pl.ds(sb * A, ACH_R)], idx_a)
        pltpu.async_copy(va2e_hbm.at[idx_a], abuf, sem).wait()
        for s in range(ACH_S):
            for col in range(D // 16):
                acc = abuf[s * A, pl.ds(col * 16, 16)]
                for a in range(1, A):
                    acc = acc + abuf[s * A + a, pl.ds(col * 16, 16)]
                sbuf[s, pl.ds(col * 16, 16)] = acc
        pltpu.sync_copy(sbuf, a_out.at[pl.ds(sb, ACH_S)])
        return carry

    lax.fori_loop(0, N_ACH, a_body, 0)


@functools.lru_cache(maxsize=1)
def _sc_gather_kernel():
    return functools.partial(
        pl.kernel,
        mesh=plsc.VectorSubcoreMesh(core_axis_name="c", subcore_axis_name="s"),
        out_type=(
            jax.ShapeDtypeStruct((BK, D), jnp.float32),
            jax.ShapeDtypeStruct((BK, D), jnp.float32),
            jax.ShapeDtypeStruct((BK, D), jnp.float32),
            jax.ShapeDtypeStruct((B, D), jnp.float32),
        ),
        scratch_types=(
            pltpu.VMEM((CH,), jnp.int32),
            pltpu.VMEM((CH,), jnp.int32),
            pltpu.VMEM((ACH_R,), jnp.int32),
            pltpu.VMEM((SELF_W,), jnp.int32),
            pltpu.VMEM((CH, D), jnp.float32),
            pltpu.VMEM((CH, D), jnp.float32),
            pltpu.VMEM((ACH_R, D), jnp.float32),
            pltpu.VMEM((ACH_S, D), jnp.float32),
            pltpu.VMEM((SELF_W, D), jnp.float32),
            pltpu.SemaphoreType.DMA,
        ),
    )(_sc_gather_body)


BB = 128
GB = B // BB


def _tc_body(n_ref, r_ref, a_ref, s_ref,
             w1a_ref, w1b_ref, w1c_ref, b1_ref, w2_ref, b2_ref,
             a1o_ref, a1u_ref, ab1_ref, a2_ref, ab2_ref, a3_ref,
             out_ref):
    f32 = jnp.float32
    n = n_ref[...]
    r = r_ref[...]
    a = a_ref[...]
    h = jnp.dot(n, w1a_ref[...], preferred_element_type=f32)
    h = h + jnp.dot(r, w1b_ref[...], preferred_element_type=f32)
    h = h + jnp.dot(a, w1c_ref[...], preferred_element_type=f32)
    h = jnp.maximum(h + b1_ref[...], 0.0)
    o = jnp.maximum(
        jnp.dot(h, w2_ref[...], preferred_element_type=f32) + b2_ref[...], 0.0)
    su = jnp.dot(s_ref[...], a1u_ref[...], preferred_element_type=f32) + ab1_ref[...]
    t = jnp.dot(o, a1o_ref[...], preferred_element_type=f32)
    t = jnp.maximum(t.reshape(BB, K, D) + su[:, None, :], 0.0).reshape(BB * K, D)
    t = jnp.maximum(
        jnp.dot(t, a2_ref[...], preferred_element_type=f32) + ab2_ref[...], 0.0)
    logits = jnp.sum(t.reshape(BB, K, D) * a3_ref[...].reshape(1, 1, D), axis=2)
    m = jnp.max(logits, axis=1, keepdims=True)
    e = jnp.exp(logits - m)
    att = e / jnp.sum(e, axis=1, keepdims=True)
    o3 = o.reshape(BB, K, D)
    acc = o3[:, 0, :] * att[:, 0:1]
    for k in range(1, K):
        acc = acc + o3[:, k, :] * att[:, k:k + 1]
    out_ref[...] = acc


def _tc_call(n_es, r_es, a_es, s_e, w1a, w1b, w1c, b1, w2, b2,
             a1o, a1u, ab1, a2w, ab2, a3v):
    row_spec = pl.BlockSpec((BB * K, D), lambda i: (i, 0))
    self_spec = pl.BlockSpec((BB, D), lambda i: (i, 0))

    def w_spec(x):
        return pl.BlockSpec(x.shape, lambda i: tuple(0 for _ in x.shape))

    return pl.pallas_call(
        _tc_body,
        grid=(GB,),
        in_specs=[row_spec, row_spec, row_spec, self_spec,
                  w_spec(w1a), w_spec(w1b), w_spec(w1c), w_spec(b1),
                  w_spec(w2), w_spec(b2), w_spec(a1o), w_spec(a1u),
                  w_spec(ab1), w_spec(a2w), w_spec(ab2), w_spec(a3v)],
        out_specs=self_spec,
        out_shape=jax.ShapeDtypeStruct((B, D), jnp.float32),
    )(n_es, r_es, a_es, s_e, w1a, w1b, w1c, b1, w2, b2,
      a1o, a1u, ab1, a2w, ab2, a3v)


def kernel(nodes, nodes_l1paths, nodes_l1n_attrs, u2e, v2e, r2e, ua2e, va2e,
           w1_w, w1_b, w2_w, w2_b, a1_w, a1_b, a2_w, a2_b, a3_w, a3_b):
    neighs = nodes_l1paths[:, :, 1].reshape(BK)
    rels = nodes_l1paths[:, :, 0].reshape(BK)
    attrs = nodes_l1n_attrs.reshape(BK * A)
    n_es, r_es, a_es, s_e = _sc_gather_kernel()(
        neighs, rels, attrs, nodes, v2e, r2e, va2e, u2e)
    return _tc_call(
        n_es, r_es, a_es, s_e,
        w1_w[0:D], w1_w[D:2 * D], w1_w[2 * D:3 * D], w1_b.reshape(1, D),
        w2_w, w2_b.reshape(1, D),
        a1_w[0:D], a1_w[D:2 * D], a1_b.reshape(1, D),
        a2_w, a2_b.reshape(1, D), a3_w.reshape(1, D))


# speedup vs baseline: 5.3210x; 1.7309x over previous
import functools

import jax
import jax.numpy as jnp
from jax import lax
from jax.experimental import pallas as pl
from jax.experimental.pallas import tpu as pltpu
from jax.experimental.pallas import tpu_sc as plsc

B, K, A, D = 1024, 32, 8, 128
BK = B * K

NC, NS = 2, 16
NW = NC * NS
SLOTS_W = BK // NW
CH = 128
N_CH = SLOTS_W // CH
ACH_S = 16
ACH_R = ACH_S * A
N_ACH = SLOTS_W // ACH_S
SELF_W = B // NW


def _sc_gather_body(neigh_hbm, rel_hbm, attr_hbm, node_hbm,
                    v2e_hbm, r2e_hbm, va2e_hbm, u2e_hbm,
                    n_out, r_out, a_out, s_out,
                    idx_n, idx_r, idx_a, idx_s,
                    bufn0, bufn1, bufr0, bufr1,
                    abuf0, abuf1, sbuf0, sbuf1, sebuf,
                    gsem, asem, wsem, awsem, ssem):
    wid = lax.axis_index("s") * NC + lax.axis_index("c")
    base = wid * SLOTS_W
    sbase = wid * SELF_W
    bufns = (bufn0, bufn1)
    bufrs = (bufr0, bufr1)
    abufs = (abuf0, abuf1)
    sbufs = (sbuf0, sbuf1)

    pltpu.sync_copy(neigh_hbm.at[pl.ds(base, SLOTS_W)], idx_n)
    pltpu.sync_copy(rel_hbm.at[pl.ds(base, SLOTS_W)], idx_r)
    pltpu.sync_copy(attr_hbm.at[pl.ds(base * A, SLOTS_W * A)], idx_a)
    pltpu.sync_copy(node_hbm.at[pl.ds(sbase, SELF_W)], idx_s)

    self_cp = pltpu.async_copy(u2e_hbm.at[idx_s], sebuf, ssem)

    def fire_nr(c, b):
        pltpu.async_copy(v2e_hbm.at[idx_n.at[pl.ds(c * CH, CH)]],
                         bufns[b], gsem)
        pltpu.async_copy(r2e_hbm.at[idx_r.at[pl.ds(c * CH, CH)]],
                         bufrs[b], gsem)

    def wait_nr(c, b):
        pltpu.make_async_copy(v2e_hbm.at[idx_n.at[pl.ds(c * CH, CH)]],
                              bufns[b], gsem).wait()
        pltpu.make_async_copy(r2e_hbm.at[idx_r.at[pl.ds(c * CH, CH)]],
                              bufrs[b], gsem).wait()

    def drain_nr_wb(b):
        pltpu.make_async_copy(bufns[b], n_out.at[pl.ds(0, CH)], wsem).wait()
        pltpu.make_async_copy(bufrs[b], r_out.at[pl.ds(0, CH)], wsem).wait()

    fire_nr(0, 0)

    def nr_iter(c2, carry):
        for b in range(2):
            c = c2 * 2 + b
            if b == 0:
                @pl.when(c2 > 0)
                def _():
                    drain_nr_wb(1)
            else:
                drain_nr_wb(0)
            if b == 0:
                fire_nr(c + 1, 1)
            else:
                @pl.when(c2 < N_CH // 2 - 1)
                def _():
                    fire_nr(c + 1, 0)
            wait_nr(c, b)
            rb = base + c * CH
            pltpu.async_copy(bufns[b], n_out.at[pl.ds(rb, CH)], wsem)
            pltpu.async_copy(bufrs[b], r_out.at[pl.ds(rb, CH)], wsem)
        return carry

    lax.fori_loop(0, N_CH // 2, nr_iter, 0)
    drain_nr_wb(1)

    def fire_a(c, b):
        pltpu.async_copy(va2e_hbm.at[idx_a.at[pl.ds(c * ACH_R, ACH_R)]],
                         abufs[b], asem)

    def wait_a(c, b):
        pltpu.make_async_copy(va2e_hbm.at[idx_a.at[pl.ds(c * ACH_R, ACH_R)]],
                              abufs[b], asem).wait()

    def drain_a_wb(b):
        pltpu.make_async_copy(sbufs[b], a_out.at[pl.ds(0, ACH_S)],
                              awsem).wait()

    fire_a(0, 0)

    def a_iter(c2, carry):
        for b in range(2):
            c = c2 * 2 + b
            if b == 0:
                fire_a(c + 1, 1)
            else:
                @pl.when(c2 < N_ACH // 2 - 1)
                def _():
                    fire_a(c + 1, 0)
            @pl.when(c2 > 0)
            def _():
                drain_a_wb(b)
            wait_a(c, b)
            abuf = abufs[b]
            sbuf = sbufs[b]

            def sum_body(s, carry2):
                for col in range(D // 16):
                    acc = abuf[s * A, pl.ds(col * 16, 16)]
                    for a in range(1, A):
                        acc = acc + abuf[s * A + a, pl.ds(col * 16, 16)]
                    sbuf[s, pl.ds(col * 16, 16)] = acc
                return carry2

            lax.fori_loop(0, ACH_S, sum_body, 0)
            sb = base + c * ACH_S
            pltpu.async_copy(sbuf, a_out.at[pl.ds(sb, ACH_S)], awsem)
        return carry

    lax.fori_loop(0, N_ACH // 2, a_iter, 0)
    drain_a_wb(0)
    drain_a_wb(1)

    self_cp.wait()
    pltpu.sync_copy(sebuf, s_out.at[pl.ds(sbase, SELF_W)])


@functools.lru_cache(maxsize=1)
def _sc_gather_kernel():
    return functools.partial(
        pl.kernel,
        mesh=plsc.VectorSubcoreMesh(core_axis_name="c", subcore_axis_name="s"),
        out_type=(
            jax.ShapeDtypeStruct((BK, D), jnp.float32),
            jax.ShapeDtypeStruct((BK, D), jnp.float32),
            jax.ShapeDtypeStruct((BK, D), jnp.float32),
            jax.ShapeDtypeStruct((B, D), jnp.float32),
        ),
        scratch_types=(
            pltpu.VMEM((SLOTS_W,), jnp.int32),
            pltpu.VMEM((SLOTS_W,), jnp.int32),
            pltpu.VMEM((SLOTS_W * A,), jnp.int32),
            pltpu.VMEM((SELF_W,), jnp.int32),
            pltpu.VMEM((CH, D), jnp.float32),
            pltpu.VMEM((CH, D), jnp.float32),
            pltpu.VMEM((CH, D), jnp.float32),
            pltpu.VMEM((CH, D), jnp.float32),
            pltpu.VMEM((ACH_R, D), jnp.float32),
            pltpu.VMEM((ACH_R, D), jnp.float32),
            pltpu.VMEM((ACH_S, D), jnp.float32),
            pltpu.VMEM((ACH_S, D), jnp.float32),
            pltpu.VMEM((SELF_W, D), jnp.float32),
            pltpu.SemaphoreType.DMA,
            pltpu.SemaphoreType.DMA,
            pltpu.SemaphoreType.DMA,
            pltpu.SemaphoreType.DMA,
            pltpu.SemaphoreType.DMA,
        ),
    )(_sc_gather_body)


BB = 128
GB = B // BB


def _tc_body(n_ref, r_ref, a_ref, s_ref,
             w1a_ref, w1b_ref, w1c_ref, b1_ref, w2_ref, b2_ref,
             a1o_ref, a1u_ref, ab1_ref, a2_ref, ab2_ref, a3_ref,
             out_ref):
    f32 = jnp.float32
    n = n_ref[...]
    r = r_ref[...]
    a = a_ref[...]
    h = jnp.dot(n, w1a_ref[...], preferred_element_type=f32)
    h = h + jnp.dot(r, w1b_ref[...], preferred_element_type=f32)
    h = h + jnp.dot(a, w1c_ref[...], preferred_element_type=f32)
    h = jnp.maximum(h + b1_ref[...], 0.0)
    o = jnp.maximum(
        jnp.dot(h, w2_ref[...], preferred_element_type=f32) + b2_ref[...], 0.0)
    su = jnp.dot(s_ref[...], a1u_ref[...], preferred_element_type=f32) + ab1_ref[...]
    t = jnp.dot(o, a1o_ref[...], preferred_element_type=f32)
    t = jnp.maximum(t.reshape(BB, K, D) + su[:, None, :], 0.0).reshape(BB * K, D)
    t = jnp.maximum(
        jnp.dot(t, a2_ref[...], preferred_element_type=f32) + ab2_ref[...], 0.0)
    logits = jnp.sum(t.reshape(BB, K, D) * a3_ref[...].reshape(1, 1, D), axis=2)
    m = jnp.max(logits, axis=1, keepdims=True)
    e = jnp.exp(logits - m)
    att = e / jnp.sum(e, axis=1, keepdims=True)
    o3 = o.reshape(BB, K, D)
    acc = o3[:, 0, :] * att[:, 0:1]
    for k in range(1, K):
        acc = acc + o3[:, k, :] * att[:, k:k + 1]
    out_ref[...] = acc


def _tc_call(n_es, r_es, a_es, s_e, w1a, w1b, w1c, b1, w2, b2,
             a1o, a1u, ab1, a2w, ab2, a3v):
    row_spec = pl.BlockSpec((BB * K, D), lambda i: (i, 0))
    self_spec = pl.BlockSpec((BB, D), lambda i: (i, 0))

    def w_spec(x):
        return pl.BlockSpec(x.shape, lambda i: tuple(0 for _ in x.shape))

    return pl.pallas_call(
        _tc_body,
        grid=(GB,),
        in_specs=[row_spec, row_spec, row_spec, self_spec,
                  w_spec(w1a), w_spec(w1b), w_spec(w1c), w_spec(b1),
                  w_spec(w2), w_spec(b2), w_spec(a1o), w_spec(a1u),
                  w_spec(ab1), w_spec(a2w), w_spec(ab2), w_spec(a3v)],
        out_specs=self_spec,
        out_shape=jax.ShapeDtypeStruct((B, D), jnp.float32),
    )(n_es, r_es, a_es, s_e, w1a, w1b, w1c, b1, w2, b2,
      a1o, a1u, ab1, a2w, ab2, a3v)


def kernel(nodes, nodes_l1paths, nodes_l1n_attrs, u2e, v2e, r2e, ua2e, va2e,
           w1_w, w1_b, w2_w, w2_b, a1_w, a1_b, a2_w, a2_b, a3_w, a3_b):
    neighs = nodes_l1paths[:, :, 1].reshape(BK)
    rels = nodes_l1paths[:, :, 0].reshape(BK)
    attrs = nodes_l1n_attrs.reshape(BK * A)
    n_es, r_es, a_es, s_e = _sc_gather_kernel()(
        neighs, rels, attrs, nodes, v2e, r2e, va2e, u2e)
    return _tc_call(
        n_es, r_es, a_es, s_e,
        w1_w[0:D], w1_w[D:2 * D], w1_w[2 * D:3 * D], w1_b.reshape(1, D),
        w2_w, w2_b.reshape(1, D),
        a1_w[0:D], a1_w[D:2 * D], a1_b.reshape(1, D),
        a2_w, a2_b.reshape(1, D), a3_w.reshape(1, D))


# speedup vs baseline: 5.9346x; 1.1153x over previous
import functools

import jax
import jax.numpy as jnp
from jax import lax
from jax.experimental import pallas as pl
from jax.experimental.pallas import tpu as pltpu
from jax.experimental.pallas import tpu_sc as plsc

B, K, A, D = 1024, 32, 8, 128
BK = B * K

NC, NS = 2, 16
NW = NC * NS
CH = 128
ACH_S = 16
ACH_R = ACH_S * A

NSPLIT = 2
BH = B // NSPLIT


def _sc_gather_body(neigh_hbm, rel_hbm, attr_hbm, node_hbm,
                    v2e_hbm, r2e_hbm, va2e_hbm, u2e_hbm,
                    n_out, r_out, a_out, s_out,
                    idx_n, idx_r, idx_a, idx_s,
                    bufn0, bufn1, bufr0, bufr1,
                    abuf0, abuf1, sbuf0, sbuf1, sebuf,
                    gsem, asem, wsem, awsem, ssem,
                    *, slots_w, self_w):
    n_ch = slots_w // CH
    n_ach = slots_w // ACH_S
    wid = lax.axis_index("s") * NC + lax.axis_index("c")
    base = wid * slots_w
    sbase = wid * self_w
    bufns = (bufn0, bufn1)
    bufrs = (bufr0, bufr1)
    abufs = (abuf0, abuf1)
    sbufs = (sbuf0, sbuf1)

    pltpu.sync_copy(neigh_hbm.at[pl.ds(base, slots_w)], idx_n)
    pltpu.sync_copy(rel_hbm.at[pl.ds(base, slots_w)], idx_r)
    pltpu.sync_copy(attr_hbm.at[pl.ds(base * A, slots_w * A)], idx_a)
    pltpu.sync_copy(node_hbm.at[pl.ds(sbase, self_w)], idx_s)

    self_cp = pltpu.async_copy(u2e_hbm.at[idx_s], sebuf, ssem)

    def fire_nr(c, b):
        pltpu.async_copy(v2e_hbm.at[idx_n.at[pl.ds(c * CH, CH)]],
                         bufns[b], gsem)
        pltpu.async_copy(r2e_hbm.at[idx_r.at[pl.ds(c * CH, CH)]],
                         bufrs[b], gsem)

    def wait_nr(c, b):
        pltpu.make_async_copy(v2e_hbm.at[idx_n.at[pl.ds(c * CH, CH)]],
                              bufns[b], gsem).wait()
        pltpu.make_async_copy(r2e_hbm.at[idx_r.at[pl.ds(c * CH, CH)]],
                              bufrs[b], gsem).wait()

    def drain_nr_wb(b):
        pltpu.make_async_copy(bufns[b], n_out.at[pl.ds(0, CH)], wsem).wait()
        pltpu.make_async_copy(bufrs[b], r_out.at[pl.ds(0, CH)], wsem).wait()

    fire_nr(0, 0)

    def nr_iter(c2, carry):
        for b in range(2):
            c = c2 * 2 + b
            if b == 0:
                @pl.when(c2 > 0)
                def _():
                    drain_nr_wb(1)
            else:
                drain_nr_wb(0)
            if b == 0:
                fire_nr(c + 1, 1)
            else:
                @pl.when(c2 < n_ch // 2 - 1)
                def _():
                    fire_nr(c + 1, 0)
            wait_nr(c, b)
            rb = base + c * CH
            pltpu.async_copy(bufns[b], n_out.at[pl.ds(rb, CH)], wsem)
            pltpu.async_copy(bufrs[b], r_out.at[pl.ds(rb, CH)], wsem)
        return carry

    lax.fori_loop(0, n_ch // 2, nr_iter, 0)
    drain_nr_wb(1)

    def fire_a(c, b):
        pltpu.async_copy(va2e_hbm.at[idx_a.at[pl.ds(c * ACH_R, ACH_R)]],
                         abufs[b], asem)

    def wait_a(c, b):
        pltpu.make_async_copy(va2e_hbm.at[idx_a.at[pl.ds(c * ACH_R, ACH_R)]],
                              abufs[b], asem).wait()

    def drain_a_wb(b):
        pltpu.make_async_copy(sbufs[b], a_out.at[pl.ds(0, ACH_S)],
                              awsem).wait()

    fire_a(0, 0)

    def a_iter(c2, carry):
        for b in range(2):
            c = c2 * 2 + b
            if b == 0:
                fire_a(c + 1, 1)
            else:
                @pl.when(c2 < n_ach // 2 - 1)
                def _():
                    fire_a(c + 1, 0)
            @pl.when(c2 > 0)
            def _():
                drain_a_wb(b)
            wait_a(c, b)
            abuf = abufs[b]
            sbuf = sbufs[b]

            def sum_body(s, carry2):
                for col in range(D // 16):
                    acc = abuf[s * A, pl.ds(col * 16, 16)]
                    for a in range(1, A):
                        acc = acc + abuf[s * A + a, pl.ds(col * 16, 16)]
                    sbuf[s, pl.ds(col * 16, 16)] = acc
                return carry2

            lax.fori_loop(0, ACH_S, sum_body, 0)
            sb = base + c * ACH_S
            pltpu.async_copy(sbuf, a_out.at[pl.ds(sb, ACH_S)], awsem)
        return carry

    lax.fori_loop(0, n_ach // 2, a_iter, 0)
    drain_a_wb(0)
    drain_a_wb(1)

    self_cp.wait()
    pltpu.sync_copy(sebuf, s_out.at[pl.ds(sbase, self_w)])


@functools.lru_cache(maxsize=2)
def _sc_gather_kernel(nb):
    slots_w = nb * K // NW
    self_w = nb // NW
    body = functools.partial(_sc_gather_body, slots_w=slots_w, self_w=self_w)
    return functools.partial(
        pl.kernel,
        mesh=plsc.VectorSubcoreMesh(core_axis_name="c", subcore_axis_name="s"),
        out_type=(
            jax.ShapeDtypeStruct((nb * K, D), jnp.float32),
            jax.ShapeDtypeStruct((nb * K, D), jnp.float32),
            jax.ShapeDtypeStruct((nb * K, D), jnp.float32),
            jax.ShapeDtypeStruct((nb, D), jnp.float32),
        ),
        scratch_types=(
            pltpu.VMEM((slots_w,), jnp.int32),
            pltpu.VMEM((slots_w,), jnp.int32),
            pltpu.VMEM((slots_w * A,), jnp.int32),
            pltpu.VMEM((self_w,), jnp.int32),
            pltpu.VMEM((CH, D), jnp.float32),
            pltpu.VMEM((CH, D), jnp.float32),
            pltpu.VMEM((CH, D), jnp.float32),
            pltpu.VMEM((CH, D), jnp.float32),
            pltpu.VMEM((ACH_R, D), jnp.float32),
            pltpu.VMEM((ACH_R, D), jnp.float32),
            pltpu.VMEM((ACH_S, D), jnp.float32),
            pltpu.VMEM((ACH_S, D), jnp.float32),
            pltpu.VMEM((self_w, D), jnp.float32),
            pltpu.SemaphoreType.DMA,
            pltpu.SemaphoreType.DMA,
            pltpu.SemaphoreType.DMA,
            pltpu.SemaphoreType.DMA,
            pltpu.SemaphoreType.DMA,
        ),
    )(body)


BB = 128


def _tc_body(n_ref, r_ref, a_ref, s_ref,
             w1a_ref, w1b_ref, w1c_ref, b1_ref, w2_ref, b2_ref,
             a1o_ref, a1u_ref, ab1_ref, a2_ref, ab2_ref, a3_ref,
             out_ref):
    f32 = jnp.float32
    n = n_ref[...]
    r = r_ref[...]
    a = a_ref[...]
    h = jnp.dot(n, w1a_ref[...], preferred_element_type=f32)
    h = h + jnp.dot(r, w1b_ref[...], preferred_element_type=f32)
    h = h + jnp.dot(a, w1c_ref[...], preferred_element_type=f32)
    h = jnp.maximum(h + b1_ref[...], 0.0)
    o = jnp.maximum(
        jnp.dot(h, w2_ref[...], preferred_element_type=f32) + b2_ref[...], 0.0)
    su = jnp.dot(s_ref[...], a1u_ref[...], preferred_element_type=f32) + ab1_ref[...]
    t = jnp.dot(o, a1o_ref[...], preferred_element_type=f32)
    t = jnp.maximum(t.reshape(BB, K, D) + su[:, None, :], 0.0).reshape(BB * K, D)
    t = jnp.maximum(
        jnp.dot(t, a2_ref[...], preferred_element_type=f32) + ab2_ref[...], 0.0)
    logits = jnp.sum(t.reshape(BB, K, D) * a3_ref[...].reshape(1, 1, D), axis=2)
    m = jnp.max(logits, axis=1, keepdims=True)
    e = jnp.exp(logits - m)
    att = e / jnp.sum(e, axis=1, keepdims=True)
    o3 = o.reshape(BB, K, D)
    acc = o3[:, 0, :] * att[:, 0:1]
    for k in range(1, K):
        acc = acc + o3[:, k, :] * att[:, k:k + 1]
    out_ref[...] = acc


def _tc_call(n_es, r_es, a_es, s_e, weights):
    nb = s_e.shape[0]
    row_spec = pl.BlockSpec((BB * K, D), lambda i: (i, 0))
    self_spec = pl.BlockSpec((BB, D), lambda i: (i, 0))

    def w_spec(x):
        return pl.BlockSpec(x.shape, lambda i: tuple(0 for _ in x.shape))

    return pl.pallas_call(
        _tc_body,
        grid=(nb // BB,),
        in_specs=[row_spec, row_spec, row_spec, self_spec]
                 + [w_spec(w) for w in weights],
        out_specs=self_spec,
        out_shape=jax.ShapeDtypeStruct((nb, D), jnp.float32),
    )(n_es, r_es, a_es, s_e, *weights)


def kernel(nodes, nodes_l1paths, nodes_l1n_attrs, u2e, v2e, r2e, ua2e, va2e,
           w1_w, w1_b, w2_w, w2_b, a1_w, a1_b, a2_w, a2_b, a3_w, a3_b):
    neighs = nodes_l1paths[:, :, 1].reshape(BK)
    rels = nodes_l1paths[:, :, 0].reshape(BK)
    attrs = nodes_l1n_attrs.reshape(BK * A)
    weights = (
        w1_w[0:D], w1_w[D:2 * D], w1_w[2 * D:3 * D], w1_b.reshape(1, D),
        w2_w, w2_b.reshape(1, D),
        a1_w[0:D], a1_w[D:2 * D], a1_b.reshape(1, D),
        a2_w, a2_b.reshape(1, D), a3_w.reshape(1, D))
    sc = _sc_gather_kernel(BH)
    gathered = []
    for p in range(NSPLIT):
        r0 = p * BH * K
        gathered.append(sc(
            lax.dynamic_slice_in_dim(neighs, r0, BH * K),
            lax.dynamic_slice_in_dim(rels, r0, BH * K),
            lax.dynamic_slice_in_dim(attrs, r0 * A, BH * K * A),
            lax.dynamic_slice_in_dim(nodes, p * BH, BH),
            v2e, r2e, va2e, u2e))
    outs = [_tc_call(n_es, r_es, a_es, s_e, weights)
            for (n_es, r_es, a_es, s_e) in gathered]
    return jnp.concatenate(outs, axis=0)


# speedup vs baseline: 6.5435x; 1.1026x over previous
import functools

import jax
import jax.numpy as jnp
from jax import lax
from jax.experimental import pallas as pl
from jax.experimental.pallas import tpu as pltpu
from jax.experimental.pallas import tpu_sc as plsc

B, K, A, D = 1024, 32, 8, 128
BK = B * K

NC, NS = 2, 16
NW = NC * NS
CH = 128
ACH_S = 16
ACH_R = ACH_S * A

NSPLIT = 2
BH = B // NSPLIT


def _sc_gather_body(neigh_hbm, rel_hbm, attr_hbm, node_hbm,
                    v2e_hbm, r2e_hbm, va2e_hbm, u2e_hbm,
                    n_out, r_out, a_out, s_out,
                    idx_n, idx_r, idx_a, idx_s,
                    bufn0, bufn1, bufr0, bufr1,
                    abuf0, abuf1, sbuf0, sbuf1, sebuf,
                    gsem, asem, wsem, awsem, ssem,
                    *, slots_w, self_w):
    n_ch = slots_w // CH
    n_ach = slots_w // ACH_S
    wid = lax.axis_index("s") * NC + lax.axis_index("c")
    base = wid * slots_w
    sbase = wid * self_w
    bufns = (bufn0, bufn1)
    bufrs = (bufr0, bufr1)
    abufs = (abuf0, abuf1)
    sbufs = (sbuf0, sbuf1)

    pltpu.sync_copy(neigh_hbm.at[pl.ds(base, slots_w)], idx_n)
    pltpu.sync_copy(rel_hbm.at[pl.ds(base, slots_w)], idx_r)
    pltpu.sync_copy(attr_hbm.at[pl.ds(base * A, slots_w * A)], idx_a)
    pltpu.sync_copy(node_hbm.at[pl.ds(sbase, self_w)], idx_s)

    self_cp = pltpu.async_copy(u2e_hbm.at[idx_s], sebuf, ssem)

    def fire_nr(c, b):
        pltpu.async_copy(v2e_hbm.at[idx_n.at[pl.ds(c * CH, CH)]],
                         bufns[b], gsem)
        pltpu.async_copy(r2e_hbm.at[idx_r.at[pl.ds(c * CH, CH)]],
                         bufrs[b], gsem)

    def wait_nr(c, b):
        pltpu.make_async_copy(v2e_hbm.at[idx_n.at[pl.ds(c * CH, CH)]],
                              bufns[b], gsem).wait()
        pltpu.make_async_copy(r2e_hbm.at[idx_r.at[pl.ds(c * CH, CH)]],
                              bufrs[b], gsem).wait()

    def drain_nr_wb(b):
        pltpu.make_async_copy(bufns[b], n_out.at[pl.ds(0, CH)], wsem).wait()
        pltpu.make_async_copy(bufrs[b], r_out.at[pl.ds(0, CH)], wsem).wait()

    fire_nr(0, 0)

    def nr_iter(c2, carry):
        for b in range(2):
            c = c2 * 2 + b
            if b == 0:
                @pl.when(c2 > 0)
                def _():
                    drain_nr_wb(1)
            else:
                drain_nr_wb(0)
            if b == 0:
                fire_nr(c + 1, 1)
            else:
                @pl.when(c2 < n_ch // 2 - 1)
                def _():
                    fire_nr(c + 1, 0)
            wait_nr(c, b)
            rb = base + c * CH
            pltpu.async_copy(bufns[b], n_out.at[pl.ds(rb, CH)], wsem)
            pltpu.async_copy(bufrs[b], r_out.at[pl.ds(rb, CH)], wsem)
        return carry

    lax.fori_loop(0, n_ch // 2, nr_iter, 0)
    drain_nr_wb(1)

    def fire_a(c, b):
        pltpu.async_copy(va2e_hbm.at[idx_a.at[pl.ds(c * ACH_R, ACH_R)]],
                         abufs[b], asem)

    def wait_a(c, b):
        pltpu.make_async_copy(va2e_hbm.at[idx_a.at[pl.ds(c * ACH_R, ACH_R)]],
                              abufs[b], asem).wait()

    def drain_a_wb(b):
        pltpu.make_async_copy(sbufs[b], a_out.at[pl.ds(0, ACH_S)],
                              awsem).wait()

    fire_a(0, 0)

    def a_iter(c2, carry):
        for b in range(2):
            c = c2 * 2 + b
            if b == 0:
                fire_a(c + 1, 1)
            else:
                @pl.when(c2 < n_ach // 2 - 1)
                def _():
                    fire_a(c + 1, 0)
            @pl.when(c2 > 0)
            def _():
                drain_a_wb(b)
            wait_a(c, b)
            abuf = abufs[b]
            sbuf = sbufs[b]

            def sum_body(s, carry2):
                for col in range(D // 16):
                    acc = abuf[s * A, pl.ds(col * 16, 16)]
                    for a in range(1, A):
                        acc = acc + abuf[s * A + a, pl.ds(col * 16, 16)]
                    sbuf[s, pl.ds(col * 16, 16)] = acc
                return carry2

            lax.fori_loop(0, ACH_S, sum_body, 0)
            sb = base + c * ACH_S
            pltpu.async_copy(sbuf, a_out.at[pl.ds(sb, ACH_S)], awsem)
        return carry

    lax.fori_loop(0, n_ach // 2, a_iter, 0)
    drain_a_wb(0)
    drain_a_wb(1)

    self_cp.wait()
    pltpu.sync_copy(sebuf, s_out.at[pl.ds(sbase, self_w)])


@functools.lru_cache(maxsize=2)
def _sc_gather_kernel(nb):
    slots_w = nb * K // NW
    self_w = nb // NW
    body = functools.partial(_sc_gather_body, slots_w=slots_w, self_w=self_w)
    return functools.partial(
        pl.kernel,
        mesh=plsc.VectorSubcoreMesh(core_axis_name="c", subcore_axis_name="s"),
        out_type=(
            jax.ShapeDtypeStruct((nb * K, D), jnp.float32),
            jax.ShapeDtypeStruct((nb * K, D), jnp.float32),
            jax.ShapeDtypeStruct((nb * K, D), jnp.float32),
            jax.ShapeDtypeStruct((nb, D), jnp.float32),
        ),
        scratch_types=(
            pltpu.VMEM((slots_w,), jnp.int32),
            pltpu.VMEM((slots_w,), jnp.int32),
            pltpu.VMEM((slots_w * A,), jnp.int32),
            pltpu.VMEM((self_w,), jnp.int32),
            pltpu.VMEM((CH, D), jnp.float32),
            pltpu.VMEM((CH, D), jnp.float32),
            pltpu.VMEM((CH, D), jnp.float32),
            pltpu.VMEM((CH, D), jnp.float32),
            pltpu.VMEM((ACH_R, D), jnp.float32),
            pltpu.VMEM((ACH_R, D), jnp.float32),
            pltpu.VMEM((ACH_S, D), jnp.float32),
            pltpu.VMEM((ACH_S, D), jnp.float32),
            pltpu.VMEM((self_w, D), jnp.float32),
            pltpu.SemaphoreType.DMA,
            pltpu.SemaphoreType.DMA,
            pltpu.SemaphoreType.DMA,
            pltpu.SemaphoreType.DMA,
            pltpu.SemaphoreType.DMA,
        ),
    )(body)


BB = 128


def _tc_body(n_ref, r_ref, a_ref, s_ref,
             w1a_ref, w1b_ref, w1c_ref, b1_ref, w2_ref, b2_ref,
             a1o_ref, a1u_ref, ab1_ref, a2_ref, ab2_ref, a3_ref,
             out_ref):
    f32 = jnp.float32
    n = n_ref[...]
    r = r_ref[...]
    a = a_ref[...]
    h = jnp.dot(n, w1a_ref[...], preferred_element_type=f32)
    h = h + jnp.dot(r, w1b_ref[...], preferred_element_type=f32)
    h = h + jnp.dot(a, w1c_ref[...], preferred_element_type=f32)
    h = jnp.maximum(h + b1_ref[...], 0.0)
    o = jnp.maximum(
        jnp.dot(h, w2_ref[...], preferred_element_type=f32) + b2_ref[...], 0.0)
    su = jnp.dot(s_ref[...], a1u_ref[...], preferred_element_type=f32) + ab1_ref[...]
    t = jnp.dot(o, a1o_ref[...], preferred_element_type=f32)
    t = jnp.maximum(t.reshape(BB, K, D) + su[:, None, :], 0.0).reshape(BB * K, D)
    t = jnp.maximum(
        jnp.dot(t, a2_ref[...], preferred_element_type=f32) + ab2_ref[...], 0.0)
    l3 = jnp.dot(t, a3_ref[...], preferred_element_type=f32).reshape(BB, K, D)
    m = jnp.max(l3, axis=1, keepdims=True)
    e = jnp.exp(l3 - m)
    att3 = e / jnp.sum(e, axis=1, keepdims=True)
    out_ref[...] = jnp.sum(o.reshape(BB, K, D) * att3, axis=1)


def _tc_call(n_es, r_es, a_es, s_e, weights):
    nb = s_e.shape[0]
    row_spec = pl.BlockSpec((BB * K, D), lambda i: (i, 0))
    self_spec = pl.BlockSpec((BB, D), lambda i: (i, 0))

    def w_spec(x):
        return pl.BlockSpec(x.shape, lambda i: tuple(0 for _ in x.shape))

    return pl.pallas_call(
        _tc_body,
        grid=(nb // BB,),
        in_specs=[row_spec, row_spec, row_spec, self_spec]
                 + [w_spec(w) for w in weights],
        out_specs=self_spec,
        out_shape=jax.ShapeDtypeStruct((nb, D), jnp.float32),
    )(n_es, r_es, a_es, s_e, *weights)


def kernel(nodes, nodes_l1paths, nodes_l1n_attrs, u2e, v2e, r2e, ua2e, va2e,
           w1_w, w1_b, w2_w, w2_b, a1_w, a1_b, a2_w, a2_b, a3_w, a3_b):
    neighs = nodes_l1paths[:, :, 1].reshape(BK)
    rels = nodes_l1paths[:, :, 0].reshape(BK)
    attrs = nodes_l1n_attrs.reshape(BK * A)
    weights = (
        w1_w[0:D], w1_w[D:2 * D], w1_w[2 * D:3 * D], w1_b.reshape(1, D),
        w2_w, w2_b.reshape(1, D),
        a1_w[0:D], a1_w[D:2 * D], a1_b.reshape(1, D),
        a2_w, a2_b.reshape(1, D), jnp.tile(a3_w, (1, D)))
    sc = _sc_gather_kernel(BH)
    gathered = []
    for p in range(NSPLIT):
        r0 = p * BH * K
        gathered.append(sc(
            lax.dynamic_slice_in_dim(neighs, r0, BH * K),
            lax.dynamic_slice_in_dim(rels, r0, BH * K),
            lax.dynamic_slice_in_dim(attrs, r0 * A, BH * K * A),
            lax.dynamic_slice_in_dim(nodes, p * BH, BH),
            v2e, r2e, va2e, u2e))
    outs = [_tc_call(n_es, r_es, a_es, s_e, weights)
            for (n_es, r_es, a_es, s_e) in gathered]
    return jnp.concatenate(outs, axis=0)
